# baseline (device time: 141810 ns/iter reference)
import jax
import jax.numpy as jnp
from jax import lax
from jax.experimental import pallas as pl
from jax.experimental.pallas import tpu as pltpu

N_DEV = 4
SQ = 256
SKV_LOCAL = 4096
HQ = 8
HKV = 2
DH = 128
DMODEL = 1024
SCALE = 0.08838834764831843
KV_CHUNK = 1024

ROWS_O = HQ * SQ


def _combine(o_a, m_a, l_a, o_b, m_b, l_b):
    m_n = jnp.maximum(m_a, m_b)
    a = jnp.exp(m_a - m_n)
    b = jnp.exp(m_b - m_n)
    return o_a * a + o_b * b, m_n, l_a * a + l_b * b


def kernel(x, Wq, Wo, K_ext, V_ext):
    x2 = x.reshape(SQ, DMODEL)
    K = jnp.transpose(K_ext[0], (1, 0, 2))
    V = jnp.transpose(V_ext[0], (1, 0, 2))

    def body(x_ref, wq_ref, wo_ref, k_ref, v_ref, out_ref,
             comm_ref, ml_ref, send_sems, recv_sems,
             ml_send_sems, ml_recv_sems):
        my = lax.axis_index("i")
        left = lax.rem(my + N_DEV - 1, N_DEV)
        right = lax.rem(my + 1, N_DEV)

        barrier_sem = pltpu.get_barrier_semaphore()
        for nbr in (left, right):
            pl.semaphore_signal(
                barrier_sem, inc=1,
                device_id=(nbr,), device_id_type=pl.DeviceIdType.MESH,
            )
        pl.semaphore_wait(barrier_sem, 2)

        xv = x_ref[:, :].astype(jnp.bfloat16)
        o_parts, m_parts, l_parts = [], [], []
        for kvh in range(HKV):
            qg = jnp.concatenate(
                [
                    jnp.dot(
                        xv,
                        wq_ref[:, (4 * kvh + g) * DH:(4 * kvh + g + 1) * DH]
                        .astype(jnp.bfloat16),
                        preferred_element_type=jnp.float32,
                    )
                    for g in range(4)
                ],
                axis=0,
            ) * SCALE
            qg = qg.astype(jnp.bfloat16)

            m = l = o = None
            for c in range(SKV_LOCAL // KV_CHUNK):
                kc = k_ref[kvh, c * KV_CHUNK:(c + 1) * KV_CHUNK, :].astype(
                    jnp.bfloat16)
                vc = v_ref[kvh, c * KV_CHUNK:(c + 1) * KV_CHUNK, :].astype(
                    jnp.bfloat16)
                s = lax.dot_general(
                    qg, kc,
                    dimension_numbers=(((1,), (1,)), ((), ())),
                    preferred_element_type=jnp.float32,
                )
                mj = jnp.max(s, axis=1, keepdims=True)
                if c == 0:
                    m_n = mj
                    p = jnp.exp(s - m_n)
                    l = jnp.sum(p, axis=1, keepdims=True)
                    o = jnp.dot(p.astype(jnp.bfloat16), vc,
                                preferred_element_type=jnp.float32)
                else:
                    m_n = jnp.maximum(m, mj)
                    alpha = jnp.exp(m - m_n)
                    p = jnp.exp(s - m_n)
                    l = l * alpha + jnp.sum(p, axis=1, keepdims=True)
                    o = o * alpha + jnp.dot(
                        p.astype(jnp.bfloat16), vc,
                        preferred_element_type=jnp.float32)
                m = m_n
            o_parts.append(o)
            m_parts.append(m)
            l_parts.append(l)

        o_acc = jnp.concatenate(o_parts, axis=0)
        m_acc = jnp.concatenate(m_parts, axis=0)
        l_acc = jnp.concatenate(l_parts, axis=0)

        comm_ref[0, :, :] = o_acc
        ml_ref[0, 0, :, :] = m_acc
        ml_ref[0, 1, :, :] = l_acc

        for h in range(N_DEV - 1):
            rdma_o = pltpu.make_async_remote_copy(
                src_ref=comm_ref.at[h],
                dst_ref=comm_ref.at[h + 1],
                send_sem=send_sems.at[h],
                recv_sem=recv_sems.at[h],
                device_id=(right,),
                device_id_type=pl.DeviceIdType.MESH,
            )
            rdma_ml = pltpu.make_async_remote_copy(
                src_ref=ml_ref.at[h],
                dst_ref=ml_ref.at[h + 1],
                send_sem=ml_send_sems.at[h],
                recv_sem=ml_recv_sems.at[h],
                device_id=(right,),
                device_id_type=pl.DeviceIdType.MESH,
            )
            rdma_o.start()
            rdma_ml.start()
            rdma_o.wait()
            rdma_ml.wait()

            o_r = comm_ref[h + 1, :, :]
            m_r = ml_ref[h + 1, 0, :, :]
            l_r = ml_ref[h + 1, 1, :, :]
            o_acc, m_acc, l_acc = _combine(o_acc, m_acc, l_acc, o_r, m_r, l_r)

        o_n = o_acc / l_acc
        attn = jnp.concatenate(
            [o_n[hh * SQ:(hh + 1) * SQ, :] for hh in range(HQ)], axis=1
        )
        out_ref[:, :] = jnp.dot(
            attn.astype(jnp.bfloat16), wo_ref[:, :].astype(jnp.bfloat16),
            preferred_element_type=jnp.float32)

    out2 = pl.pallas_call(
        body,
        out_shape=jax.ShapeDtypeStruct((SQ, DMODEL), jnp.float32),
        in_specs=[pl.BlockSpec(memory_space=pltpu.VMEM)] * 5,
        out_specs=pl.BlockSpec(memory_space=pltpu.VMEM),
        scratch_shapes=[
            pltpu.VMEM((N_DEV, ROWS_O, 128), jnp.float32),
            pltpu.VMEM((N_DEV, 2, ROWS_O, 1), jnp.float32),
            pltpu.SemaphoreType.DMA((N_DEV - 1,)),
            pltpu.SemaphoreType.DMA((N_DEV - 1,)),
            pltpu.SemaphoreType.DMA((N_DEV - 1,)),
            pltpu.SemaphoreType.DMA((N_DEV - 1,)),
        ],
        compiler_params=pltpu.CompilerParams(
            collective_id=0,
            vmem_limit_bytes=100 * 1024 * 1024,
        ),
    )(x2, Wq, Wo, K, V)

    return out2.reshape(1, SQ, DMODEL)


# device time: 39850 ns/iter; 3.5586x vs baseline; 3.5586x over previous
import os

import jax
import jax.numpy as jnp
from jax import lax
from jax.experimental import pallas as pl
from jax.experimental.pallas import tpu as pltpu

N_DEV = 4
N_STEPS = 2
SQ = 256
SKV_LOCAL = 4096
HQ = 8
HKV = 2
GRP = HQ // HKV
DH = 128
DMODEL = 1024
SCALE = 0.08838834764831843
KV_CHUNK = 1024
NQ = GRP * SQ

GROUP_ROWS = 144

_SKIP_COMM = os.environ.get("SKIP_COMM", "0") == "1"


def kernel(x, Wq, Wo, K_ext, V_ext):
    x2 = x.reshape(SQ, DMODEL)
    K = K_ext[0]
    V = V_ext[0]

    def body(x_ref, wq_ref, wo_ref, k_ref, v_ref, out_ref,
             send_ref, recv_ref, kv_vmem, send_sems, recv_sems,
             copy_sems):
        my = lax.axis_index("i")
        partners = [jnp.bitwise_xor(my, 1), jnp.bitwise_xor(my, 2)]

        barrier_sem = pltpu.get_barrier_semaphore()
        for nbr in partners:
            pl.semaphore_signal(
                barrier_sem, inc=1,
                device_id=(nbr,), device_id_type=pl.DeviceIdType.MESH,
            )
        pl.semaphore_wait(barrier_sem, 2)

        def pack(step, g, o, l):
            send_ref[step, g, 0:DH, :] = o.astype(jnp.bfloat16)
            send_ref[step, g, DH:DH + 1, :] = l.astype(jnp.bfloat16)

        def exchange(step, g):
            r = pltpu.make_async_remote_copy(
                src_ref=send_ref.at[step, g],
                dst_ref=recv_ref.at[step, g],
                send_sem=send_sems.at[step, g],
                recv_sem=recv_sems.at[step, g],
                device_id=(partners[step],),
                device_id_type=pl.DeviceIdType.MESH,
            )
            r.start()
            return r

        rdmas = {}

        copies = {}
        for t, hbm in ((0, k_ref), (1, v_ref)):
            for kvh in range(HKV):
                cp = pltpu.make_async_copy(
                    hbm.at[:, kvh, :], kv_vmem.at[t, kvh],
                    copy_sems.at[t, kvh],
                )
                cp.start()
                copies[(t, kvh)] = cp

        xv = x_ref[:, :].astype(jnp.bfloat16)
        o_acc, l_acc = [], []

        def combine(g, step):
            o_r = recv_ref[step, g, 0:DH, :].astype(jnp.float32)
            l_r = recv_ref[step, g, DH:DH + 1, :].astype(jnp.float32)
            o_acc[g] = o_acc[g] + o_r
            l_acc[g] = l_acc[g] + l_r

        def project(g):
            attn_t = jnp.concatenate(
                [
                    (o_acc[g] / l_acc[g])[:, q * SQ:(q + 1) * SQ]
                    for q in range(GRP)
                ],
                axis=0,
            ).astype(jnp.bfloat16)
            return lax.dot_general(
                attn_t,
                wo_ref[g * GRP * DH:(g + 1) * GRP * DH, :]
                .astype(jnp.bfloat16),
                dimension_numbers=(((0,), (0,)), ((), ())),
                preferred_element_type=jnp.float32,
            )

        for kvh in range(HKV):
            qg = jnp.concatenate(
                [
                    jnp.dot(
                        xv,
                        wq_ref[:, (GRP * kvh + g) * DH:(GRP * kvh + g + 1) * DH]
                        .astype(jnp.bfloat16),
                        preferred_element_type=jnp.float32,
                    )
                    for g in range(GRP)
                ],
                axis=0,
            ) * SCALE
            qg = qg.astype(jnp.bfloat16)
            copies[(0, kvh)].wait()
            copies[(1, kvh)].wait()

            m = l = o = None
            for c in range(SKV_LOCAL // KV_CHUNK):
                if kvh == 1 and c == 2 and not _SKIP_COMM:
                    rdmas[(0, 0)].wait_recv()
                    combine(0, 0)
                    pack(1, 0, o_acc[0], l_acc[0])
                    rdmas[(1, 0)] = exchange(1, 0)
                rows = pl.ds(c * KV_CHUNK, KV_CHUNK)
                kc = kv_vmem[0, kvh, rows, :].astype(jnp.bfloat16)
                vc = kv_vmem[1, kvh, rows, :].astype(jnp.bfloat16)
                s = lax.dot_general(
                    kc, qg,
                    dimension_numbers=(((1,), (1,)), ((), ())),
                    preferred_element_type=jnp.float32,
                )
                s = s.astype(jnp.bfloat16)
                mj = jnp.max(s, axis=0, keepdims=True)
                if c == 0:
                    m_n = mj
                    p = jnp.exp(s - m_n)
                    l = jnp.sum(p, axis=0, keepdims=True,
                                dtype=jnp.float32)
                    o = lax.dot_general(
                        vc, p,
                        dimension_numbers=(((0,), (0,)), ((), ())),
                        preferred_element_type=jnp.float32,
                    )
                else:
                    m_n = jnp.maximum(m, mj)
                    alpha = jnp.exp((m - m_n).astype(jnp.float32))
                    p = jnp.exp(s - m_n)
                    l = l * alpha + jnp.sum(p, axis=0, keepdims=True,
                                            dtype=jnp.float32)
                    o = o * alpha + lax.dot_general(
                        vc, p,
                        dimension_numbers=(((0,), (0,)), ((), ())),
                        preferred_element_type=jnp.float32,
                    )
                m = m_n
            em = jnp.exp(m.astype(jnp.float32))
            o_acc.append(o * em)
            l_acc.append(l * em)
            if not _SKIP_COMM:
                pack(0, kvh, o_acc[kvh], l_acc[kvh])
                rdmas[(0, kvh)] = exchange(0, kvh)

        out_parts = []
        if not _SKIP_COMM:
            rdmas[(1, 0)].wait_recv()
            combine(0, 1)
        out_parts.append(project(0))
        if not _SKIP_COMM:
            rdmas[(0, 1)].wait_recv()
            combine(1, 0)
            pack(1, 1, o_acc[1], l_acc[1])
            rdmas[(1, 1)] = exchange(1, 1)
            rdmas[(1, 1)].wait_recv()
            combine(1, 1)
        out_parts.append(project(1))
        out_ref[:, :] = out_parts[0] + out_parts[1]

        for key in rdmas:
            rdmas[key].wait_send()

    out2 = pl.pallas_call(
        body,
        out_shape=jax.ShapeDtypeStruct((SQ, DMODEL), jnp.float32),
        in_specs=[pl.BlockSpec(memory_space=pltpu.VMEM)] * 3
        + [pl.BlockSpec(memory_space=pltpu.MemorySpace.HBM)] * 2,
        out_specs=pl.BlockSpec(memory_space=pltpu.VMEM),
        scratch_shapes=[
            pltpu.VMEM((N_STEPS, HKV, GROUP_ROWS, NQ), jnp.bfloat16),
            pltpu.VMEM((N_STEPS, HKV, GROUP_ROWS, NQ), jnp.bfloat16),
            pltpu.VMEM((2, HKV, SKV_LOCAL, DH), jnp.float32),
            pltpu.SemaphoreType.DMA((N_STEPS, HKV)),
            pltpu.SemaphoreType.DMA((N_STEPS, HKV)),
            pltpu.SemaphoreType.DMA((2, HKV)),
        ],
        compiler_params=pltpu.CompilerParams(
            collective_id=0,
            vmem_limit_bytes=100 * 1024 * 1024,
        ),
    )(x2, Wq, Wo, K, V)

    return out2.reshape(1, SQ, DMODEL)
